# Initial kernel scaffold; baseline (speedup 1.0000x reference)
import jax
import jax.numpy as jnp
from jax.experimental import pallas as pl


def kernel(atomic_numbers, edge_distance, edge_index, edge_envelope_weight, wigner_inv, source_emb, target_emb, W0, b0, g0, be0, W1, b1, g1, be1, W2, b2):
    raise NotImplementedError("write your pallas kernel here")


# speedup vs baseline: 9.7414x; 9.7414x over previous
import functools

import jax
import jax.numpy as jnp
import numpy as np
from jax import lax
from jax.experimental import pallas as pl
from jax.experimental.pallas import tpu as pltpu
from jax.experimental.pallas import tpu_sc as plsc

N = 10000
E = 320000
LMAX = 2
NCH = 16
SPH = (LMAX + 1) ** 2
OUT = SPH * NCH
D_DIST = 128
HID = 64
MAX_ELEM = 90
RESCALE = 32.0

NC = 2
NS = 16
NW = NC * NS
EPW = E // NW
CK = 80
NCHUNK = EPW // CK
NPAD = 10240
ROWS = NPAD // NS

_mesh = plsc.VectorSubcoreMesh(core_axis_name="c", subcore_axis_name="s")
_sc_params = pltpu.CompilerParams(needs_layout_passes=False)


@functools.partial(
    pl.kernel,
    out_type=(jax.ShapeDtypeStruct((E,), jnp.int32),
              jax.ShapeDtypeStruct((E,), jnp.int32)),
    mesh=_mesh,
    scratch_types=[
        pltpu.VMEM((N,), jnp.int32),
        pltpu.VMEM((EPW,), jnp.int32),
        pltpu.VMEM((EPW,), jnp.int32),
    ],
    compiler_params=_sc_params,
)
def _gather_els(an_hbm, sidx_hbm, tidx_hbm, souts_hbm, touts_hbm, an_v, idx_v, el_v):
    wid = lax.axis_index("s") * NC + lax.axis_index("c")
    base = wid * EPW
    pltpu.sync_copy(an_hbm, an_v)
    for in_hbm, out_hbm in ((sidx_hbm, souts_hbm), (tidx_hbm, touts_hbm)):
        pltpu.sync_copy(in_hbm.at[pl.ds(base, EPW)], idx_v)

        def body(k, carry):
            idx = idx_v[pl.ds(k * 16, 16)]
            el_v[pl.ds(k * 16, 16)] = plsc.load_gather(an_v, [idx])
            return carry

        lax.fori_loop(0, EPW // 16, body, 0)
        pltpu.sync_copy(el_v, out_hbm.at[pl.ds(base, EPW)])


BB = 1280
NB = E // BB


def _dense_body(ed_ref, sel_ref, tel_ref, env_ref, wig_ref,
                w0d_ref, ps_ref, pt_ref, b0_ref, g0_ref, be0_ref,
                w1_ref, b1_ref, g1_ref, be1_ref, w2_ref, b2_ref,
                s_ref, g_ref, h_ref, outa_ref, outb_ref):
    f32 = jnp.float32
    hi = jax.lax.Precision.HIGHEST

    def mm(a, b):
        return jnp.dot(a, b, preferred_element_type=f32, precision=hi)

    ed = ed_ref[...]
    iota = lax.broadcasted_iota(jnp.int32, (BB, 128), 1)
    ohs = (sel_ref[...] == iota).astype(f32)
    oht = (tel_ref[...] == iota).astype(f32)
    h = mm(ed, w0d_ref[...]) + mm(ohs, ps_ref[...]) + mm(oht, pt_ref[...]) + b0_ref[...]

    def ln_silu(x, g, b):
        mu = jnp.mean(x, axis=1, keepdims=True)
        xc = x - mu
        var = jnp.mean(xc * xc, axis=1, keepdims=True)
        y = xc * lax.rsqrt(var + 1e-5) * g + b
        return y * jax.nn.sigmoid(y)

    a1 = ln_silu(h, g0_ref[...], be0_ref[...])
    h2 = mm(a1, w1_ref[...]) + b1_ref[...]
    a2 = ln_silu(h2, g1_ref[...], be1_ref[...])
    r = (mm(a2, w2_ref[...]) + b2_ref[...]) * env_ref[...]

    wigj = mm(wig_ref[...], s_ref[...])
    acc = jnp.zeros((BB, OUT), f32)
    for j in range(LMAX + 1):
        wj = wigj[:, j * SPH:(j + 1) * SPH]
        xj = r[:, j * NCH:(j + 1) * NCH]
        acc = acc + mm(wj, g_ref[...]) * mm(xj, h_ref[...])
    outa_ref[...] = acc[:, :128]
    outb_ref[...] = jnp.concatenate(
        [acc[:, 128:], jnp.zeros((BB, 256 - OUT), f32)], axis=1)


def _make_dense():
    full = lambda shape: pl.BlockSpec(shape, lambda i: (0,) * len(shape))
    row = lambda w: pl.BlockSpec((BB, w), lambda i: (i, 0))
    return pl.pallas_call(
        _dense_body,
        grid=(NB,),
        in_specs=[
            row(D_DIST), row(1), row(1), row(1), row(SPH * SPH),
            full((128, HID)), full((128, HID)), full((128, HID)),
            full((1, HID)), full((1, HID)), full((1, HID)),
            full((HID, HID)), full((1, HID)), full((1, HID)), full((1, HID)),
            full((HID, (LMAX + 1) * NCH)), full((1, (LMAX + 1) * NCH)),
            full((SPH * SPH, (LMAX + 1) * SPH)),
            full((SPH, OUT)), full((NCH, OUT)),
        ],
        out_specs=[pl.BlockSpec((BB, 128), lambda i: (i, 0)),
                   pl.BlockSpec((BB, 128), lambda i: (i, 0))],
        out_shape=[jax.ShapeDtypeStruct((E, 128), jnp.float32),
                   jax.ShapeDtypeStruct((E, 128), jnp.float32)],
        compiler_params=pltpu.CompilerParams(dimension_semantics=("arbitrary",)),
    )


_dense = _make_dense()


EPS = E // NS


@functools.partial(
    pl.kernel,
    out_type=jax.ShapeDtypeStruct((NC, NPAD, 128), jnp.float32),
    mesh=_mesh,
    scratch_types=[
        pltpu.VMEM_SHARED((NPAD, 128), jnp.float32),
        pltpu.VMEM((CK, 128), jnp.float32),
        pltpu.VMEM((CK,), jnp.int32),
    ],
    compiler_params=_sc_params,
)
def _scatter(xa_hbm, xb_hbm, tgt_hbm, zer_hbm, out_hbm, acc, vals_v, idx_v):
    cid = lax.axis_index("c")
    sid = lax.axis_index("s")
    pltpu.sync_copy(zer_hbm, acc.at[pl.ds(sid * ROWS, ROWS)])
    plsc.subcore_barrier()
    base = sid * EPS

    def sweep(src_hbm):
        def chunk(k, carry):
            off = base + k * CK
            pltpu.sync_copy(tgt_hbm.at[pl.ds(off, CK)], idx_v)
            pltpu.sync_copy(src_hbm.at[pl.ds(off, CK)], vals_v)
            pltpu.sync_copy(vals_v, acc.at[idx_v], add=True)
            return carry

        lax.fori_loop(0, EPS // CK, chunk, 0)

    @pl.when(cid == 0)
    def _():
        sweep(xa_hbm)

    @pl.when(cid == 1)
    def _():
        sweep(xb_hbm)

    plsc.subcore_barrier()
    r0 = sid * ROWS
    pltpu.sync_copy(acc.at[pl.ds(r0, ROWS)], out_hbm.at[cid, pl.ds(r0, ROWS)])


def kernel(atomic_numbers, edge_distance, edge_index, edge_envelope_weight,
           wigner_inv, source_emb, target_emb, W0, b0, g0, be0, W1, b1, g1,
           be1, W2, b2):
    f32 = jnp.float32
    an = atomic_numbers.astype(jnp.int32)
    eidx = edge_index.astype(jnp.int32)

    src_el, tgt_el = _gather_els(an, eidx[0], eidx[1])
    sel = src_el.reshape(E, 1)
    tel = tgt_el.reshape(E, 1)

    w0d = W0[:D_DIST]
    semb = jnp.zeros((128, HID), f32).at[:MAX_ELEM].set(source_emb)
    temb = jnp.zeros((128, HID), f32).at[:MAX_ELEM].set(target_emb)
    ps = semb @ W0[D_DIST:D_DIST + HID]
    pt = temb @ W0[D_DIST + HID:]
    r1 = lambda v: v.reshape(1, -1)

    s_np = np.zeros((SPH * SPH, (LMAX + 1) * SPH), np.float32)
    for i in range(SPH):
        for j in range(LMAX + 1):
            s_np[i * SPH + j, j * SPH + i] = 1.0
    g_np = np.zeros((SPH, OUT), np.float32)
    for i in range(SPH):
        g_np[i, i * NCH:(i + 1) * NCH] = 1.0
    h_np = np.zeros((NCH, OUT), np.float32)
    for c in range(NCH):
        for i in range(SPH):
            h_np[c, i * NCH + c] = 1.0

    xa, xb = _dense(
        edge_distance, sel, tel, edge_envelope_weight.astype(f32),
        wigner_inv.reshape(E, SPH * SPH),
        w0d, ps, pt, r1(b0), r1(g0), r1(be0),
        W1, r1(b1), r1(g1), r1(be1), W2, r1(b2),
        jnp.asarray(s_np), jnp.asarray(g_np), jnp.asarray(h_np),
    )

    zer = jnp.zeros((ROWS, 128), f32)
    partials = _scatter(xa, xb, eidx[1], zer)
    out = jnp.concatenate([partials[0, :N], partials[1, :N, :OUT - 128]],
                          axis=1) * (1.0 / RESCALE)
    return out.reshape(N, SPH, NCH)


# speedup vs baseline: 23.4728x; 2.4096x over previous
import functools

import jax
import jax.numpy as jnp
import numpy as np
from jax import lax
from jax.experimental import pallas as pl
from jax.experimental.pallas import tpu as pltpu
from jax.experimental.pallas import tpu_sc as plsc

N = 10000
E = 320000
LMAX = 2
NCH = 16
SPH = (LMAX + 1) ** 2
OUT = SPH * NCH
D_DIST = 128
HID = 64
MAX_ELEM = 90
RESCALE = 32.0

NC = 2
NS = 16
NW = NC * NS
EPW = E // NW
CK = 80
NCHUNK = EPW // CK
NPAD = 10240
ROWS = NPAD // NS

_mesh = plsc.VectorSubcoreMesh(core_axis_name="c", subcore_axis_name="s")
_sc_params = pltpu.CompilerParams(needs_layout_passes=False)


@functools.partial(
    pl.kernel,
    out_type=(jax.ShapeDtypeStruct((E,), jnp.int32),
              jax.ShapeDtypeStruct((E,), jnp.int32)),
    mesh=_mesh,
    scratch_types=[
        pltpu.VMEM((N,), jnp.int32),
        pltpu.VMEM((EPW,), jnp.int32),
        pltpu.VMEM((EPW,), jnp.int32),
    ],
    compiler_params=_sc_params,
)
def _gather_els(an_hbm, sidx_hbm, tidx_hbm, souts_hbm, touts_hbm, an_v, idx_v, el_v):
    wid = lax.axis_index("s") * NC + lax.axis_index("c")
    base = wid * EPW
    pltpu.sync_copy(an_hbm, an_v)
    for in_hbm, out_hbm in ((sidx_hbm, souts_hbm), (tidx_hbm, touts_hbm)):
        pltpu.sync_copy(in_hbm.at[pl.ds(base, EPW)], idx_v)

        def body(k, carry):
            idx = idx_v[pl.ds(k * 16, 16)]
            el_v[pl.ds(k * 16, 16)] = plsc.load_gather(an_v, [idx])
            return carry

        lax.fori_loop(0, EPW // 16, body, 0)
        pltpu.sync_copy(el_v, out_hbm.at[pl.ds(base, EPW)])


BB = 1280
NB = E // BB


def _dense_body(ed_ref, sel_ref, tel_ref, env_ref, wig_ref,
                w0d_ref, ps_ref, pt_ref, b0_ref, g0_ref, be0_ref,
                w1_ref, b1_ref, g1_ref, be1_ref, w2_ref, b2_ref,
                s_ref, g_ref, h_ref, outa_ref, outb_ref):
    f32 = jnp.float32

    def mm(a, b):
        return jnp.dot(a, b, preferred_element_type=f32)

    ed = ed_ref[...]
    iota = lax.broadcasted_iota(jnp.int32, (BB, 128), 1)
    ohs = (sel_ref[...] == iota).astype(f32)
    oht = (tel_ref[...] == iota).astype(f32)
    h = mm(ed, w0d_ref[...]) + mm(ohs, ps_ref[...]) + mm(oht, pt_ref[...]) + b0_ref[...]

    def ln_silu(x, g, b):
        mu = jnp.mean(x, axis=1, keepdims=True)
        xc = x - mu
        var = jnp.mean(xc * xc, axis=1, keepdims=True)
        y = xc * lax.rsqrt(var + 1e-5) * g + b
        return y * jax.nn.sigmoid(y)

    a1 = ln_silu(h, g0_ref[...], be0_ref[...])
    h2 = mm(a1, w1_ref[...]) + b1_ref[...]
    a2 = ln_silu(h2, g1_ref[...], be1_ref[...])
    r = (mm(a2, w2_ref[...]) + b2_ref[...]) * env_ref[...]

    wigj = mm(wig_ref[...], s_ref[...])
    acc = jnp.zeros((BB, OUT), f32)
    for j in range(LMAX + 1):
        wj = wigj[:, j * SPH:(j + 1) * SPH]
        xj = r[:, j * NCH:(j + 1) * NCH]
        acc = acc + mm(wj, g_ref[...]) * mm(xj, h_ref[...])
    outa_ref[...] = acc[:, :128]
    outb_ref[...] = jnp.concatenate(
        [acc[:, 128:], jnp.zeros((BB, 256 - OUT), f32)], axis=1)


def _make_dense():
    full = lambda shape: pl.BlockSpec(shape, lambda i: (0,) * len(shape))
    row = lambda w: pl.BlockSpec((BB, w), lambda i: (i, 0))
    return pl.pallas_call(
        _dense_body,
        grid=(NB,),
        in_specs=[
            row(D_DIST), row(1), row(1), row(1), row(SPH * SPH),
            full((128, HID)), full((128, HID)), full((128, HID)),
            full((1, HID)), full((1, HID)), full((1, HID)),
            full((HID, HID)), full((1, HID)), full((1, HID)), full((1, HID)),
            full((HID, (LMAX + 1) * NCH)), full((1, (LMAX + 1) * NCH)),
            full((SPH * SPH, (LMAX + 1) * SPH)),
            full((SPH, OUT)), full((NCH, OUT)),
        ],
        out_specs=[pl.BlockSpec((BB, 128), lambda i: (i, 0)),
                   pl.BlockSpec((BB, 128), lambda i: (i, 0))],
        out_shape=[jax.ShapeDtypeStruct((E, 128), jnp.float32),
                   jax.ShapeDtypeStruct((E, 128), jnp.float32)],
        compiler_params=pltpu.CompilerParams(dimension_semantics=("arbitrary",)),
    )


_dense = _make_dense()


EPS = E // NS


@functools.partial(
    pl.kernel,
    out_type=jax.ShapeDtypeStruct((NC, NPAD, 128), jnp.float32),
    mesh=_mesh,
    scratch_types=[
        pltpu.VMEM_SHARED((NPAD, 128), jnp.float32),
        pltpu.VMEM((CK, 128), jnp.float32),
        pltpu.VMEM((CK,), jnp.int32),
    ],
    compiler_params=_sc_params,
)
def _scatter(xa_hbm, xb_hbm, tgt_hbm, zer_hbm, out_hbm, acc, vals_v, idx_v):
    cid = lax.axis_index("c")
    sid = lax.axis_index("s")
    pltpu.sync_copy(zer_hbm, acc.at[pl.ds(sid * ROWS, ROWS)])
    plsc.subcore_barrier()
    base = sid * EPS

    def sweep(src_hbm):
        def chunk(k, carry):
            off = base + k * CK
            pltpu.sync_copy(tgt_hbm.at[pl.ds(off, CK)], idx_v)
            pltpu.sync_copy(src_hbm.at[pl.ds(off, CK)], vals_v)
            pltpu.sync_copy(vals_v, acc.at[idx_v], add=True)
            return carry

        lax.fori_loop(0, EPS // CK, chunk, 0)

    @pl.when(cid == 0)
    def _():
        sweep(xa_hbm)

    @pl.when(cid == 1)
    def _():
        sweep(xb_hbm)

    plsc.subcore_barrier()
    r0 = sid * ROWS
    pltpu.sync_copy(acc.at[pl.ds(r0, ROWS)], out_hbm.at[cid, pl.ds(r0, ROWS)])


def kernel(atomic_numbers, edge_distance, edge_index, edge_envelope_weight,
           wigner_inv, source_emb, target_emb, W0, b0, g0, be0, W1, b1, g1,
           be1, W2, b2):
    f32 = jnp.float32
    an = atomic_numbers.astype(jnp.int32)
    eidx = edge_index.astype(jnp.int32)

    src_el, tgt_el = _gather_els(an, eidx[0], eidx[1])
    sel = src_el.reshape(E, 1)
    tel = tgt_el.reshape(E, 1)

    w0d = W0[:D_DIST]
    semb = jnp.zeros((128, HID), f32).at[:MAX_ELEM].set(source_emb)
    temb = jnp.zeros((128, HID), f32).at[:MAX_ELEM].set(target_emb)
    ps = semb @ W0[D_DIST:D_DIST + HID]
    pt = temb @ W0[D_DIST + HID:]
    r1 = lambda v: v.reshape(1, -1)

    s_np = np.zeros((SPH * SPH, (LMAX + 1) * SPH), np.float32)
    for i in range(SPH):
        for j in range(LMAX + 1):
            s_np[i * SPH + j, j * SPH + i] = 1.0
    g_np = np.zeros((SPH, OUT), np.float32)
    for i in range(SPH):
        g_np[i, i * NCH:(i + 1) * NCH] = 1.0
    h_np = np.zeros((NCH, OUT), np.float32)
    for c in range(NCH):
        for i in range(SPH):
            h_np[c, i * NCH + c] = 1.0

    xa, xb = _dense(
        edge_distance, sel, tel, edge_envelope_weight.astype(f32),
        wigner_inv.reshape(E, SPH * SPH),
        w0d, ps, pt, r1(b0), r1(g0), r1(be0),
        W1, r1(b1), r1(g1), r1(be1), W2, r1(b2),
        jnp.asarray(s_np), jnp.asarray(g_np), jnp.asarray(h_np),
    )

    zer = jnp.zeros((ROWS, 128), f32)
    partials = _scatter(xa, xb, eidx[1], zer)
    out = jnp.concatenate([partials[0, :N], partials[1, :N, :OUT - 128]],
                          axis=1) * (1.0 / RESCALE)
    return out.reshape(N, SPH, NCH)


# speedup vs baseline: 28.2992x; 1.2056x over previous
import functools

import jax
import jax.numpy as jnp
import numpy as np
from jax import lax
from jax.experimental import pallas as pl
from jax.experimental.pallas import tpu as pltpu
from jax.experimental.pallas import tpu_sc as plsc

N = 10000
E = 320000
LMAX = 2
NCH = 16
SPH = (LMAX + 1) ** 2
OUT = SPH * NCH
D_DIST = 128
HID = 64
MAX_ELEM = 90
RESCALE = 32.0

NC = 2
NS = 16
NW = NC * NS
EPW = E // NW
CK = 80
NCHUNK = EPW // CK
NPAD = 10240
ROWS = NPAD // NS

_mesh = plsc.VectorSubcoreMesh(core_axis_name="c", subcore_axis_name="s")
_sc_params = pltpu.CompilerParams(needs_layout_passes=False)


@functools.partial(
    pl.kernel,
    out_type=(jax.ShapeDtypeStruct((E,), jnp.float32),
              jax.ShapeDtypeStruct((E,), jnp.float32)),
    mesh=_mesh,
    scratch_types=[
        pltpu.VMEM((N,), jnp.int32),
        pltpu.VMEM((EPW,), jnp.int32),
        pltpu.VMEM((EPW,), jnp.float32),
    ],
    compiler_params=_sc_params,
)
def _gather_els(an_hbm, sidx_hbm, tidx_hbm, souts_hbm, touts_hbm, an_v, idx_v, el_v):
    wid = lax.axis_index("s") * NC + lax.axis_index("c")
    base = wid * EPW
    pltpu.sync_copy(an_hbm, an_v)
    for in_hbm, out_hbm in ((sidx_hbm, souts_hbm), (tidx_hbm, touts_hbm)):
        pltpu.sync_copy(in_hbm.at[pl.ds(base, EPW)], idx_v)

        def body(k, carry):
            idx = idx_v[pl.ds(k * 16, 16)]
            el = plsc.load_gather(an_v, [idx])
            el_v[pl.ds(k * 16, 16)] = el.astype(jnp.float32)
            return carry

        lax.fori_loop(0, EPW // 16, body, 0)
        pltpu.sync_copy(el_v, out_hbm.at[pl.ds(base, EPW)])


BB = 1280
NB = E // BB


def _dense_body(ed_ref, sel_ref, tel_ref, env_ref, wig_ref,
                w0d_ref, ps_ref, pt_ref, b0_ref, g0_ref, be0_ref,
                w1_ref, b1_ref, g1_ref, be1_ref, w2_ref, b2_ref,
                g_ref, h_ref, outa_ref, outb_ref):
    f32 = jnp.float32

    def mm(a, b):
        return jnp.dot(a, b, preferred_element_type=f32)

    def mmt(a, b):
        return lax.dot_general(a, b, (((0,), (0,)), ((), ())),
                               preferred_element_type=f32)

    ed = ed_ref[...]
    iota = lax.broadcasted_iota(jnp.int32, (128, BB), 0).astype(f32)
    selr = sel_ref[...].reshape(1, BB)
    telr = tel_ref[...].reshape(1, BB)
    ohst = (iota == selr).astype(f32)
    ohtt = (iota == telr).astype(f32)
    h = mm(ed, w0d_ref[...]) + mmt(ohst, ps_ref[...]) + mmt(ohtt, pt_ref[...]) + b0_ref[...]

    def ln_silu(x, g, b):
        mu = jnp.mean(x, axis=1, keepdims=True)
        xc = x - mu
        var = jnp.mean(xc * xc, axis=1, keepdims=True)
        y = xc * lax.rsqrt(var + 1e-5) * g + b
        return y * jax.nn.sigmoid(y)

    a1 = ln_silu(h, g0_ref[...], be0_ref[...])
    h2 = mm(a1, w1_ref[...]) + b1_ref[...]
    a2 = ln_silu(h2, g1_ref[...], be1_ref[...])
    r = mm(a2, w2_ref[...]) + b2_ref[...]

    envr = env_ref[...].reshape(1, BB)
    acc = jnp.zeros((BB, OUT), f32)
    for j in range(LMAX + 1):
        wjt = wig_ref[pl.ds(j * SPH, SPH), :] * envr
        xj = r[:, j * NCH:(j + 1) * NCH]
        acc = acc + mmt(wjt, g_ref[...]) * mm(xj, h_ref[...])
    outa_ref[...] = acc[:, :128]
    outb_ref[...] = jnp.concatenate(
        [acc[:, 128:], jnp.zeros((BB, 256 - OUT), f32)], axis=1)


def _make_dense():
    full = lambda shape: pl.BlockSpec(shape, lambda i: (0,) * len(shape))
    row = lambda w: pl.BlockSpec((BB, w), lambda i: (i, 0))
    return pl.pallas_call(
        _dense_body,
        grid=(NB,),
        in_specs=[
            row(D_DIST),
            pl.BlockSpec((1, 1, BB), lambda i: (i, 0, 0)),
            pl.BlockSpec((1, 1, BB), lambda i: (i, 0, 0)),
            pl.BlockSpec((1, 1, BB), lambda i: (i, 0, 0)),
            pl.BlockSpec(((LMAX + 1) * SPH, BB), lambda i: (0, i)),
            full((128, HID)), full((128, HID)), full((128, HID)),
            full((1, HID)), full((1, HID)), full((1, HID)),
            full((HID, HID)), full((1, HID)), full((1, HID)), full((1, HID)),
            full((HID, (LMAX + 1) * NCH)), full((1, (LMAX + 1) * NCH)),
            full((SPH, OUT)), full((NCH, OUT)),
        ],
        out_specs=[pl.BlockSpec((BB, 128), lambda i: (i, 0)),
                   pl.BlockSpec((BB, 128), lambda i: (i, 0))],
        out_shape=[jax.ShapeDtypeStruct((E, 128), jnp.float32),
                   jax.ShapeDtypeStruct((E, 128), jnp.float32)],
        compiler_params=pltpu.CompilerParams(dimension_semantics=("arbitrary",)),
    )


_dense = _make_dense()


EPS = E // NS


@functools.partial(
    pl.kernel,
    out_type=jax.ShapeDtypeStruct((NC, NPAD, 128), jnp.float32),
    mesh=_mesh,
    scratch_types=[
        pltpu.VMEM_SHARED((NPAD, 128), jnp.float32),
        pltpu.VMEM((CK, 128), jnp.float32),
        pltpu.VMEM((CK,), jnp.int32),
    ],
    compiler_params=_sc_params,
)
def _scatter(xa_hbm, xb_hbm, tgt_hbm, zer_hbm, out_hbm, acc, vals_v, idx_v):
    cid = lax.axis_index("c")
    sid = lax.axis_index("s")
    pltpu.sync_copy(zer_hbm, acc.at[pl.ds(sid * ROWS, ROWS)])
    plsc.subcore_barrier()
    base = sid * EPS

    def sweep(src_hbm):
        def chunk(k, carry):
            off = base + k * CK
            pltpu.sync_copy(tgt_hbm.at[pl.ds(off, CK)], idx_v)
            pltpu.sync_copy(src_hbm.at[pl.ds(off, CK)], vals_v)
            pltpu.sync_copy(vals_v, acc.at[idx_v], add=True)
            return carry

        lax.fori_loop(0, EPS // CK, chunk, 0)

    @pl.when(cid == 0)
    def _():
        sweep(xa_hbm)

    @pl.when(cid == 1)
    def _():
        sweep(xb_hbm)

    plsc.subcore_barrier()
    r0 = sid * ROWS
    pltpu.sync_copy(acc.at[pl.ds(r0, ROWS)], out_hbm.at[cid, pl.ds(r0, ROWS)])


def kernel(atomic_numbers, edge_distance, edge_index, edge_envelope_weight,
           wigner_inv, source_emb, target_emb, W0, b0, g0, be0, W1, b1, g1,
           be1, W2, b2):
    f32 = jnp.float32
    an = atomic_numbers.astype(jnp.int32)
    eidx = edge_index.astype(jnp.int32)

    sel, tel = _gather_els(an, eidx[0], eidx[1])

    w0d = W0[:D_DIST]
    semb = jnp.zeros((128, HID), f32).at[:MAX_ELEM].set(source_emb)
    temb = jnp.zeros((128, HID), f32).at[:MAX_ELEM].set(target_emb)
    ps = semb @ W0[D_DIST:D_DIST + HID]
    pt = temb @ W0[D_DIST + HID:]
    r1 = lambda v: v.reshape(1, -1)

    g_np = np.zeros((SPH, OUT), np.float32)
    for i in range(SPH):
        g_np[i, i * NCH:(i + 1) * NCH] = 1.0
    h_np = np.zeros((NCH, OUT), np.float32)
    for c in range(NCH):
        for i in range(SPH):
            h_np[c, i * NCH + c] = 1.0

    wigjt = jnp.transpose(wigner_inv[:, :, :LMAX + 1], (2, 1, 0)).reshape(
        (LMAX + 1) * SPH, E)
    xa, xb = _dense(
        edge_distance, sel.reshape(NB, 1, BB), tel.reshape(NB, 1, BB),
        edge_envelope_weight.reshape(NB, 1, BB),
        wigjt,
        w0d, ps, pt, r1(b0), r1(g0), r1(be0),
        W1, r1(b1), r1(g1), r1(be1), W2, r1(b2),
        jnp.asarray(g_np), jnp.asarray(h_np),
    )

    zer = jnp.zeros((ROWS, 128), f32)
    partials = _scatter(xa, xb, eidx[1], zer)
    out = jnp.concatenate([partials[0, :N], partials[1, :N, :OUT - 128]],
                          axis=1) * (1.0 / RESCALE)
    return out.reshape(N, SPH, NCH)


# speedup vs baseline: 33.8636x; 1.1966x over previous
import functools

import jax
import jax.numpy as jnp
import numpy as np
from jax import lax
from jax.experimental import pallas as pl
from jax.experimental.pallas import tpu as pltpu
from jax.experimental.pallas import tpu_sc as plsc

N = 10000
E = 320000
LMAX = 2
NCH = 16
SPH = (LMAX + 1) ** 2
OUT = SPH * NCH
D_DIST = 128
HID = 64
MAX_ELEM = 90
RESCALE = 32.0

NC = 2
NS = 16
NW = NC * NS
EPW = E // NW
CK = 80
NCHUNK = EPW // CK
NPAD = 10240
ROWS = NPAD // NS

_mesh = plsc.VectorSubcoreMesh(core_axis_name="c", subcore_axis_name="s")
_sc_params = pltpu.CompilerParams(needs_layout_passes=False)


@functools.partial(
    pl.kernel,
    out_type=(jax.ShapeDtypeStruct((E,), jnp.float32),
              jax.ShapeDtypeStruct((E,), jnp.float32)),
    mesh=_mesh,
    scratch_types=[
        pltpu.VMEM((N,), jnp.int32),
        pltpu.VMEM((EPW,), jnp.int32),
        pltpu.VMEM((EPW,), jnp.float32),
    ],
    compiler_params=_sc_params,
)
def _gather_els(an_hbm, sidx_hbm, tidx_hbm, souts_hbm, touts_hbm, an_v, idx_v, el_v):
    wid = lax.axis_index("s") * NC + lax.axis_index("c")
    base = wid * EPW
    pltpu.sync_copy(an_hbm, an_v)
    for in_hbm, out_hbm in ((sidx_hbm, souts_hbm), (tidx_hbm, touts_hbm)):
        pltpu.sync_copy(in_hbm.at[pl.ds(base, EPW)], idx_v)

        def body(k, carry):
            idx = idx_v[pl.ds(k * 16, 16)]
            el = plsc.load_gather(an_v, [idx])
            el_v[pl.ds(k * 16, 16)] = el.astype(jnp.float32)
            return carry

        lax.fori_loop(0, EPW // 16, body, 0)
        pltpu.sync_copy(el_v, out_hbm.at[pl.ds(base, EPW)])


BB = 1280
NB = E // BB


def _dense_body(ed_ref, sel_ref, tel_ref, env_ref, wig_ref,
                w0d_ref, ps_ref, pt_ref, b0_ref, g0_ref, be0_ref,
                w1_ref, b1_ref, g1_ref, be1_ref, w2_ref, b2_ref,
                g_ref, h_ref, outa_ref, outb_ref):
    f32 = jnp.float32

    def mm(a, b):
        return jnp.dot(a, b, preferred_element_type=f32)

    def mmt(a, b):
        return lax.dot_general(a, b, (((0,), (0,)), ((), ())),
                               preferred_element_type=f32)

    ed = ed_ref[...]
    iota = lax.broadcasted_iota(jnp.int32, (128, BB), 0).astype(f32)
    selr = sel_ref[...].reshape(1, BB)
    telr = tel_ref[...].reshape(1, BB)
    ohst = (iota == selr).astype(f32)
    ohtt = (iota == telr).astype(f32)
    h = mm(ed, w0d_ref[...]) + mmt(ohst, ps_ref[...]) + mmt(ohtt, pt_ref[...]) + b0_ref[...]

    def ln_silu(x, g, b):
        mu = jnp.mean(x, axis=1, keepdims=True)
        xc = x - mu
        var = jnp.mean(xc * xc, axis=1, keepdims=True)
        y = xc * lax.rsqrt(var + 1e-5) * g + b
        return y * jax.nn.sigmoid(y)

    a1 = ln_silu(h, g0_ref[...], be0_ref[...])
    h2 = mm(a1, w1_ref[...]) + b1_ref[...]
    a2 = ln_silu(h2, g1_ref[...], be1_ref[...])
    r = mm(a2, w2_ref[...]) + b2_ref[...]

    envr = env_ref[...].reshape(1, BB)
    acc = jnp.zeros((BB, OUT), f32)
    for j in range(LMAX + 1):
        wjt = jnp.concatenate(
            [wig_ref[i, j, :].reshape(1, BB) for i in range(SPH)],
            axis=0) * envr
        xj = r[:, j * NCH:(j + 1) * NCH]
        acc = acc + mmt(wjt, g_ref[...]) * mm(xj, h_ref[...])
    outa_ref[...] = acc[:, :128]
    outb_ref[...] = jnp.concatenate(
        [acc[:, 128:], jnp.zeros((BB, 256 - OUT), f32)], axis=1)


def _make_dense():
    full = lambda shape: pl.BlockSpec(shape, lambda i: (0,) * len(shape))
    row = lambda w: pl.BlockSpec((BB, w), lambda i: (i, 0))
    return pl.pallas_call(
        _dense_body,
        grid=(NB,),
        in_specs=[
            row(D_DIST),
            pl.BlockSpec((1, 1, BB), lambda i: (i, 0, 0)),
            pl.BlockSpec((1, 1, BB), lambda i: (i, 0, 0)),
            pl.BlockSpec((1, 1, BB), lambda i: (i, 0, 0)),
            pl.BlockSpec((SPH, SPH, BB), lambda i: (0, 0, i)),
            full((128, HID)), full((128, HID)), full((128, HID)),
            full((1, HID)), full((1, HID)), full((1, HID)),
            full((HID, HID)), full((1, HID)), full((1, HID)), full((1, HID)),
            full((HID, (LMAX + 1) * NCH)), full((1, (LMAX + 1) * NCH)),
            full((SPH, OUT)), full((NCH, OUT)),
        ],
        out_specs=[pl.BlockSpec((BB, 128), lambda i: (i, 0)),
                   pl.BlockSpec((BB, 128), lambda i: (i, 0))],
        out_shape=[jax.ShapeDtypeStruct((E, 128), jnp.float32),
                   jax.ShapeDtypeStruct((E, 128), jnp.float32)],
        compiler_params=pltpu.CompilerParams(dimension_semantics=("arbitrary",)),
    )


_dense = _make_dense()


EPS = E // NS


@functools.partial(
    pl.kernel,
    out_type=jax.ShapeDtypeStruct((NC, NPAD, 128), jnp.float32),
    mesh=_mesh,
    scratch_types=[
        pltpu.VMEM_SHARED((NPAD, 128), jnp.float32),
        pltpu.VMEM((CK, 128), jnp.float32),
        pltpu.VMEM((CK,), jnp.int32),
    ],
    compiler_params=_sc_params,
)
def _scatter(xa_hbm, xb_hbm, tgt_hbm, zer_hbm, out_hbm, acc, vals_v, idx_v):
    cid = lax.axis_index("c")
    sid = lax.axis_index("s")
    pltpu.sync_copy(zer_hbm, acc.at[pl.ds(sid * ROWS, ROWS)])
    plsc.subcore_barrier()
    base = sid * EPS

    def sweep(src_hbm):
        def chunk(k, carry):
            off = base + k * CK
            pltpu.sync_copy(tgt_hbm.at[pl.ds(off, CK)], idx_v)
            pltpu.sync_copy(src_hbm.at[pl.ds(off, CK)], vals_v)
            pltpu.sync_copy(vals_v, acc.at[idx_v], add=True)
            return carry

        lax.fori_loop(0, EPS // CK, chunk, 0)

    @pl.when(cid == 0)
    def _():
        sweep(xa_hbm)

    @pl.when(cid == 1)
    def _():
        sweep(xb_hbm)

    plsc.subcore_barrier()
    r0 = sid * ROWS
    pltpu.sync_copy(acc.at[pl.ds(r0, ROWS)], out_hbm.at[cid, pl.ds(r0, ROWS)])


def kernel(atomic_numbers, edge_distance, edge_index, edge_envelope_weight,
           wigner_inv, source_emb, target_emb, W0, b0, g0, be0, W1, b1, g1,
           be1, W2, b2):
    f32 = jnp.float32
    an = atomic_numbers.astype(jnp.int32)
    eidx = edge_index.astype(jnp.int32)

    sel, tel = _gather_els(an, eidx[0], eidx[1])

    w0d = W0[:D_DIST]
    semb = jnp.zeros((128, HID), f32).at[:MAX_ELEM].set(source_emb)
    temb = jnp.zeros((128, HID), f32).at[:MAX_ELEM].set(target_emb)
    ps = semb @ W0[D_DIST:D_DIST + HID]
    pt = temb @ W0[D_DIST + HID:]
    r1 = lambda v: v.reshape(1, -1)

    g_np = np.zeros((SPH, OUT), np.float32)
    for i in range(SPH):
        g_np[i, i * NCH:(i + 1) * NCH] = 1.0
    h_np = np.zeros((NCH, OUT), np.float32)
    for c in range(NCH):
        for i in range(SPH):
            h_np[c, i * NCH + c] = 1.0

    wigjt = jnp.transpose(wigner_inv, (1, 2, 0))
    xa, xb = _dense(
        edge_distance, sel.reshape(NB, 1, BB), tel.reshape(NB, 1, BB),
        edge_envelope_weight.reshape(NB, 1, BB),
        wigjt,
        w0d, ps, pt, r1(b0), r1(g0), r1(be0),
        W1, r1(b1), r1(g1), r1(be1), W2, r1(b2),
        jnp.asarray(g_np), jnp.asarray(h_np),
    )

    zer = jnp.zeros((ROWS, 128), f32)
    partials = _scatter(xa, xb, eidx[1], zer)
    out = jnp.concatenate([partials[0, :N], partials[1, :N, :OUT - 128]],
                          axis=1) * (1.0 / RESCALE)
    return out.reshape(N, SPH, NCH)
